# baseline (speedup 1.0000x reference)
import jax
import jax.numpy as jnp
from jax import lax
from jax.experimental import pallas as pl
from jax.experimental.pallas import tpu as pltpu
from jax.experimental.pallas import tpu_sc as plsc

_NUM_CORES = 2
_NUM_SUBCORES = 16
_NUM_WORKERS = _NUM_CORES * _NUM_SUBCORES
_CHUNK = 128


def _emb_gather(idx_hbm, table_hbm, out_hbm, idx_v, rows_v, sem):
    wid = lax.axis_index("s") * _NUM_CORES + lax.axis_index("c")
    nchunk = idx_v.shape[0]
    pltpu.sync_copy(idx_hbm.at[wid], idx_v)
    copies = [
        pltpu.async_copy(table_hbm.at[idx_v.at[j]], rows_v.at[j], sem)
        for j in range(nchunk)
    ]
    for c in copies:
        c.wait()
    pltpu.sync_copy(rows_v, out_hbm.at[wid])


def kernel(indices, table):
    batch = indices.shape[0]
    dim = table.shape[1]
    nchunk = batch // (_NUM_WORKERS * _CHUNK)
    idx3 = indices.astype(jnp.int32).reshape(_NUM_WORKERS, nchunk, _CHUNK)
    mesh = plsc.VectorSubcoreMesh(core_axis_name="c", subcore_axis_name="s")
    run = pl.kernel(
        _emb_gather,
        mesh=mesh,
        out_type=jax.ShapeDtypeStruct((_NUM_WORKERS, nchunk, _CHUNK, dim),
                                      jnp.float32),
        scratch_types=[
            pltpu.VMEM((nchunk, _CHUNK), jnp.int32),
            pltpu.VMEM((nchunk, _CHUNK, dim), jnp.float32),
            pltpu.SemaphoreType.DMA,
        ],
        compiler_params=pltpu.CompilerParams(use_tc_tiling_on_sc=False),
    )
    out = run(idx3, table)
    return out.reshape(batch, dim)


# speedup vs baseline: 1.6376x; 1.6376x over previous
import jax
import jax.numpy as jnp
from jax import lax
from jax.experimental import pallas as pl
from jax.experimental.pallas import tpu as pltpu
from jax.experimental.pallas import tpu_sc as plsc

_NUM_CORES = 2
_NUM_SUBCORES = 16
_NUM_WORKERS = _NUM_CORES * _NUM_SUBCORES
_LANES = 16


def _emb_gather(idx_hbm, table_hbm, out_hbm, idx_v, out_v, sem):
    wid = lax.axis_index("s") * _NUM_CORES + lax.axis_index("c")
    b_per_w = idx_v.shape[0]
    pltpu.sync_copy(idx_hbm.at[wid], idx_v)

    def group(g, carry):
        grp = idx_v[pl.ds(g * _LANES, _LANES)]
        copies = [
            pltpu.async_copy(table_hbm.at[pl.ds(grp[j], 1)],
                             out_v.at[pl.ds(g * _LANES + j, 1)], sem)
            for j in range(_LANES)
        ]
        for c in copies:
            c.wait()
        return carry

    lax.fori_loop(0, b_per_w // _LANES, group, 0)
    pltpu.sync_copy(out_v, out_hbm.at[wid])


def kernel(indices, table):
    batch = indices.shape[0]
    dim = table.shape[1]
    b_per_w = batch // _NUM_WORKERS
    idx2 = indices.astype(jnp.int32).reshape(_NUM_WORKERS, b_per_w)
    mesh = plsc.VectorSubcoreMesh(core_axis_name="c", subcore_axis_name="s")
    run = pl.kernel(
        _emb_gather,
        mesh=mesh,
        out_type=jax.ShapeDtypeStruct((_NUM_WORKERS, b_per_w, dim),
                                      jnp.float32),
        scratch_types=[
            pltpu.VMEM((b_per_w,), jnp.int32),
            pltpu.VMEM((b_per_w, dim), jnp.float32),
            pltpu.SemaphoreType.DMA,
        ],
    )
    out = run(idx2, table)
    return out.reshape(batch, dim)


# speedup vs baseline: 1.6408x; 1.0020x over previous
import jax
import jax.numpy as jnp
from jax import lax
from jax.experimental import pallas as pl
from jax.experimental.pallas import tpu as pltpu
from jax.experimental.pallas import tpu_sc as plsc

_NUM_CORES = 2
_NUM_SUBCORES = 16
_NUM_WORKERS = _NUM_CORES * _NUM_SUBCORES
_LANES = 16


def _emb_gather(idx_hbm, table_hbm, out_hbm, idx_v, out_v, sem):
    wid = lax.axis_index("s") * _NUM_CORES + lax.axis_index("c")
    b_per_w = idx_v.shape[0]
    pltpu.sync_copy(idx_hbm.at[wid], idx_v)

    def group(g, carry):
        grp = idx_v[pl.ds(g * _LANES, _LANES)]
        copies = [
            pltpu.async_copy(table_hbm.at[pl.ds(grp[j], 1)],
                             out_v.at[pl.ds(g * _LANES + j, 1)], sem)
            for j in range(_LANES)
        ]
        for c in copies:
            c.wait()
        return carry

    lax.fori_loop(0, b_per_w // _LANES, group, 0)
    pltpu.sync_copy(out_v, out_hbm.at[wid])


def kernel(indices, table):
    batch = indices.shape[0]
    dim = table.shape[1]
    b_per_w = batch // _NUM_WORKERS
    idx2 = indices.astype(jnp.int32).reshape(_NUM_WORKERS, b_per_w)
    mesh = plsc.VectorSubcoreMesh(core_axis_name="c", subcore_axis_name="s")
    run = pl.kernel(
        _emb_gather,
        mesh=mesh,
        out_type=jax.ShapeDtypeStruct((_NUM_WORKERS, b_per_w, dim),
                                      jnp.float32),
        scratch_types=[
            pltpu.VMEM((b_per_w,), jnp.int32),
            pltpu.VMEM((b_per_w, dim), jnp.float32),
            pltpu.SemaphoreType.DMA,
        ],
        compiler_params=pltpu.CompilerParams(use_tc_tiling_on_sc=True),
    )
    out = run(idx2, table)
    return out.reshape(batch, dim)


# speedup vs baseline: 3.0478x; 1.8575x over previous
import jax
import jax.numpy as jnp
from jax import lax
from jax.experimental import pallas as pl
from jax.experimental.pallas import tpu as pltpu
from jax.experimental.pallas import tpu_sc as plsc

_NUM_CORES = 2
_NUM_SUBCORES = 16
_NUM_WORKERS = _NUM_CORES * _NUM_SUBCORES
_LANES = 16
_RING = 8


def _emb_gather(idx_hbm, tab_hbm, out_hbm, idx_v, tile_v, out_v, *sems):
    wid = lax.axis_index("s") * _NUM_CORES + lax.axis_index("c")
    b_per_w = out_v.shape[1]
    dim = out_v.shape[0]
    n_waves = b_per_w // _RING
    pltpu.sync_copy(idx_hbm.at[wid], idx_v.at[pl.ds(0, b_per_w)])

    def issue(i, slot):
        off = pl.multiple_of((i >> 7) * 128, 128)
        return pltpu.async_copy(
            tab_hbm.at[:, pl.ds(off, 128)],
            tile_v.at[pl.ds(slot * dim, dim)], sems[slot])

    def drain(slot):
        pltpu.make_async_copy(
            tab_hbm.at[:, pl.ds(0, 128)],
            tile_v.at[pl.ds(slot * dim, dim)], sems[slot]).wait()

    grp0 = idx_v[pl.ds(0, _LANES)]
    for j in range(_RING):
        issue(grp0[j], j)

    d_base = lax.iota(jnp.int32, _LANES)

    def wave(g, carry):
        sel_grp = idx_v[pl.ds(g * _RING, _LANES)]
        iss_grp = idx_v[pl.ds((g + 1) * _RING, _LANES)]
        for j in range(_RING):
            drain(j)
            lane = sel_grp[j] & 127
            lane_vec = jnp.full((_LANES,), lane, jnp.int32)
            b_vec = jnp.full((_LANES,), g * _RING + j, jnp.int32)
            for dg in range(dim // _LANES):
                src_rows = d_base + (j * dim + dg * _LANES)
                r = plsc.load_gather(tile_v, [src_rows, lane_vec])
                plsc.store_scatter(out_v, [d_base + dg * _LANES, b_vec], r)

            @pl.when(g < n_waves - 1)
            def _():
                issue(iss_grp[j], j)

        return carry

    lax.fori_loop(0, n_waves, wave, 0)
    pltpu.sync_copy(out_v,
                    out_hbm.at[:, pl.ds(wid * b_per_w, b_per_w)])


def kernel(indices, table):
    batch = indices.shape[0]
    dim = table.shape[1]
    b_per_w = batch // _NUM_WORKERS
    idx2 = indices.astype(jnp.int32).reshape(_NUM_WORKERS, b_per_w)
    mesh = plsc.VectorSubcoreMesh(core_axis_name="c", subcore_axis_name="s")
    run = pl.kernel(
        _emb_gather,
        mesh=mesh,
        out_type=jax.ShapeDtypeStruct((dim, batch), jnp.float32),
        scratch_types=[
            pltpu.VMEM((b_per_w + 2 * _LANES,), jnp.int32),
            pltpu.VMEM((_RING * dim, 128), jnp.float32),
            pltpu.VMEM((dim, b_per_w), jnp.float32),
        ] + [pltpu.SemaphoreType.DMA] * _RING,
        compiler_params=pltpu.CompilerParams(use_tc_tiling_on_sc=True,
                                             needs_layout_passes=False),
    )
    out_t = run(idx2, table.T)
    return out_t.T


# speedup vs baseline: 3.8846x; 1.2746x over previous
import jax
import jax.numpy as jnp
from jax import lax
from jax.experimental import pallas as pl
from jax.experimental.pallas import tpu as pltpu
from jax.experimental.pallas import tpu_sc as plsc

_NUM_CORES = 2
_NUM_SUBCORES = 16
_NUM_WORKERS = _NUM_CORES * _NUM_SUBCORES
_L = 16
_NCOLS = 7813
_COLS_PER_W = 245
_CAP = 1024
_NSB = 16
_RING = 4
_ROWRING = 16


def _emb_gather(idx_hbm, tab_hbm, out_hbm,
                idx_v, hits_v, buck_v, match_v, tile_v, row_v, cnt_s, *sems):
    colsems = sems[:_RING]
    rowsem = sems[_RING]
    wid = lax.axis_index("s") * _NUM_CORES + lax.axis_index("c")
    base_col = wid * _COLS_PER_W
    ncol = jnp.minimum(_COLS_PER_W, _NCOLS - base_col)
    lo = base_col * 128
    hi = lo + ncol * 128
    pltpu.sync_copy(idx_hbm, idx_v)

    iota = lax.iota(jnp.int32, _L)
    lane0 = iota == 0

    ncolp = ((ncol + _RING - 1) >> 2) << 2

    def issue_col(c, slot):
        cc = jnp.minimum(base_col + c, _NCOLS - 1)
        off = pl.multiple_of(cc * 128, 128)
        pltpu.async_copy(tab_hbm.at[:, pl.ds(off, 128)],
                         tile_v.at[pl.ds(slot * 64, 64)], colsems[slot])

    def wait_col(slot):
        pltpu.make_async_copy(tab_hbm.at[:, pl.ds(0, 128)],
                              tile_v.at[pl.ds(slot * 64, 64)],
                              colsems[slot]).wait()

    def pass_body(state):
        p, rowcnt, _ = state
        for sb in range(_NSB):
            cnt_s[sb] = 0
        lo_rank = p * _CAP
        hi_rank = lo_rank + _CAP

        def scan4(v4, carry):
            h, off = carry
            for u in range(4):
                v = v4 * 4 + u
                x = idx_v[pl.ds(v * _L, _L)]
                m = (x >= lo) & (x < hi)
                cs = plsc.cumsum(m.astype(jnp.int32))
                r = cs + h
                sub = m & (r > lo_rank) & (r <= hi_rank)
                pk = (x - lo) * 16384 + (iota + v * _L)
                plsc.store_compressed(hits_v.at[pl.ds(off, _L)], pk, mask=sub)
                off = off + plsc.all_reduce_population_count(sub)[0]
                h = h + cs[_L - 1]
            return h, off

        total, emitted = lax.fori_loop(0, 16384 // _L // 4, scan4, (0, 0))

        def buck(j, carry):
            pkv = plsc.load_gather(hits_v, [jnp.full((_L,), j, jnp.int32)])
            sb = pkv[0] >> 25
            n = cnt_s[sb]
            plsc.store_scatter(
                buck_v, [jnp.full((_L,), sb * _CAP + n, jnp.int32)],
                pkv, mask=lane0)
            cnt_s[sb] = n + 1
            return carry

        lax.fori_loop(0, emitted, buck, 0)

        for s in range(_RING):
            issue_col(jnp.int32(s), s)

        def wave(c4, rc):
            for s in range(_RING):
                c = c4 * _RING + s
                wait_col(s)
                sb = c >> 4
                nb = cnt_s[sb]

                def mscan(v, mo, sb=sb, c=c, nb=nb):
                    pkv = buck_v[pl.ds(sb * _CAP + v * _L, _L)]
                    cm = ((pkv >> 21) == c) & ((iota + v * _L) < nb)
                    plsc.store_compressed(match_v.at[pl.ds(mo, _L)], pkv, mask=cm)
                    return mo + plsc.all_reduce_population_count(cm)[0]

                nm = lax.fori_loop(0, (nb + _L - 1) >> 4, mscan, 0)

                def proc(m, rc2, s=s):
                    pkv = plsc.load_gather(match_v,
                                           [jnp.full((_L,), m, jnp.int32)])
                    pk = pkv[0]
                    lane = (pk >> 14) & 127
                    b = pk & 16383
                    lane_vec = jnp.full((_L,), lane, jnp.int32)
                    rs = lax.rem(rc2, _ROWRING)

                    @pl.when((rs == 0) & (rc2 > 0))
                    def _():
                        for _k in range(_ROWRING):
                            pltpu.make_async_copy(
                                row_v.at[pl.ds(0, 128)], out_hbm.at[0],
                                rowsem).wait()

                    for dg in range(4):
                        r = plsc.load_gather(
                            tile_v, [s * 64 + dg * _L + iota, lane_vec])
                        row_v[pl.ds(rs * 128 + dg * _L, _L)] = r
                    pltpu.async_copy(row_v.at[pl.ds(rs * 128, 128)],
                                     out_hbm.at[b], rowsem)
                    return rc2 + 1

                rc = lax.fori_loop(0, nm, proc, rc)

                @pl.when(c + _RING < ncolp)
                def _(c=c, s=s):
                    issue_col(c + _RING, s)

            return rc

        rowcnt = lax.fori_loop(0, ncolp >> 2, wave, rowcnt)
        return p + 1, rowcnt, total > hi_rank

    state = lax.while_loop(lambda s: s[2], pass_body,
                           (jnp.int32(0), jnp.int32(0), jnp.bool_(True)))
    nrows = state[1]

    pending = jnp.where(
        nrows > 0, nrows - ((nrows - 1) // _ROWRING) * _ROWRING, 0)

    def drain(i, carry):
        pltpu.make_async_copy(row_v.at[pl.ds(0, 128)], out_hbm.at[0],
                              rowsem).wait()
        return carry

    lax.fori_loop(0, pending, drain, 0)


def kernel(indices, table):
    batch = indices.shape[0]
    dim = table.shape[1]
    idx1 = indices.astype(jnp.int32)
    mesh = plsc.VectorSubcoreMesh(core_axis_name="c", subcore_axis_name="s")
    run = pl.kernel(
        _emb_gather,
        mesh=mesh,
        out_type=jax.ShapeDtypeStruct((batch, 128), jnp.float32),
        scratch_types=[
            pltpu.VMEM((batch,), jnp.int32),
            pltpu.VMEM((_CAP + _L,), jnp.int32),
            pltpu.VMEM((_NSB * _CAP,), jnp.int32),
            pltpu.VMEM((_CAP + _L,), jnp.int32),
            pltpu.VMEM((_RING * 64, 128), jnp.float32),
            pltpu.VMEM((_ROWRING * 128,), jnp.float32),
            pltpu.SMEM((_NSB,), jnp.int32),
        ] + [pltpu.SemaphoreType.DMA] * (_RING + 1),
        compiler_params=pltpu.CompilerParams(use_tc_tiling_on_sc=True,
                                             needs_layout_passes=False),
    )
    return run(idx1, table.T)[:, :dim]
